# Initial kernel scaffold; baseline (speedup 1.0000x reference)
import jax
import jax.numpy as jnp
from jax.experimental import pallas as pl


def kernel(embeddings, relation_weights, pos_src_interacts, pos_dst_interacts, neg_src_interacts, neg_dst_interacts, pos_src_regulates, pos_dst_regulates, neg_src_regulates, neg_dst_regulates):
    raise NotImplementedError("write your pallas kernel here")


# speedup vs baseline: 7.4518x; 7.4518x over previous
import functools

import jax
import jax.numpy as jnp
from jax import lax
from jax.experimental import pallas as pl
from jax.experimental.pallas import tpu as pltpu
from jax.experimental.pallas import tpu_sc as plsc

N = 100000
D = 128
E = 320000
NUM_REL = 2

NC = 2
NS = 16
NW = NC * NS
C = 80
ROWS = E // C
WROWS = ROWS // NW
GROUPS = C // 16


def _sc_body(emb, rel, sidx0, didx0, sidx1, didx1, sidx2, didx2, sidx3,
             didx3, out, sidx_v, didx_v, srows0, drows0, srows1, drows1,
             rel_v, scores_v, sem_g):
    wid = lax.axis_index("s") * NC + lax.axis_index("c")

    pltpu.sync_copy(rel, rel_v)

    iota = lax.iota(jnp.int32, 16)
    _gdn = lax.GatherDimensionNumbers(offset_dims=(),
                                      collapsed_slice_dims=(0,),
                                      start_index_map=(0,))

    def _perm(x, perm):
        return lax.gather(x, perm[:, None], _gdn, slice_sizes=(1,),
                          mode=lax.GatherScatterMode.PROMISE_IN_BOUNDS)

    perms = [iota ^ sh for sh in (8, 4, 2, 1)]
    src_refs = (sidx0, sidx1, sidx2, sidx3)
    dst_refs = (didx0, didx1, didx2, didx3)

    def fire(ci, sbuf, dbuf, b):
        pltpu.async_copy(emb.at[sidx_v.at[ci]], sbuf, sem_g.at[b])
        pltpu.async_copy(emb.at[didx_v.at[ci]], dbuf, sem_g.at[b])

    def drain(ci, sbuf, dbuf, b):
        pltpu.make_async_copy(emb.at[sidx_v.at[ci]], sbuf,
                              sem_g.at[b]).wait()
        pltpu.make_async_copy(emb.at[didx_v.at[ci]], dbuf,
                              sem_g.at[b]).wait()

    for t in range(4):
        rel_row = t // 2
        pltpu.sync_copy(src_refs[t].at[wid], sidx_v)
        pltpu.sync_copy(dst_refs[t].at[wid], didx_v)

        rvs = [rel_v[rel_row, pl.ds(k * 16, 16)] for k in range(D // 16)]

        def compute(ci, sbuf, dbuf):
            def gbody(g, carry):
                def ebody(j, res):
                    e = g * 16 + j
                    acc = (sbuf[e, pl.ds(0, 16)] * dbuf[e, pl.ds(0, 16)]
                           * rvs[0])
                    for k in range(1, D // 16):
                        acc = acc + (sbuf[e, pl.ds(k * 16, 16)]
                                     * dbuf[e, pl.ds(k * 16, 16)]
                                     * rvs[k])
                    for p in perms:
                        acc = acc + _perm(acc, p)
                    return jnp.where(iota == j, acc, res)

                res = lax.fori_loop(0, 16, ebody,
                                    jnp.zeros((16,), jnp.float32),
                                    unroll=4)
                scores_v[ci, pl.ds(g * 16, 16)] = res
                return carry

            lax.fori_loop(0, GROUPS, gbody, 0)

        fire(0, srows0, drows0, 0)

        def pair_body(i, carry):
            c0 = 2 * i
            fire(c0 + 1, srows1, drows1, 1)
            drain(c0, srows0, drows0, 0)
            compute(c0, srows0, drows0)
            fire(c0 + 2, srows0, drows0, 0)
            drain(c0 + 1, srows1, drows1, 1)
            compute(c0 + 1, srows1, drows1)
            return carry

        lax.fori_loop(0, (WROWS - 1) // 2, pair_body, 0)
        drain(WROWS - 1, srows0, drows0, 0)
        compute(WROWS - 1, srows0, drows0)

        pltpu.sync_copy(scores_v, out.at[t, wid])


@functools.partial(
    pl.kernel,
    out_type=jax.ShapeDtypeStruct((4, NW, WROWS, C), jnp.float32),
    mesh=plsc.VectorSubcoreMesh(core_axis_name="c", subcore_axis_name="s",
                                num_cores=NC, num_subcores=NS),
    scratch_types=[
        pltpu.VMEM((WROWS, C), jnp.int32),
        pltpu.VMEM((WROWS, C), jnp.int32),
        pltpu.VMEM((C, D), jnp.float32),
        pltpu.VMEM((C, D), jnp.float32),
        pltpu.VMEM((C, D), jnp.float32),
        pltpu.VMEM((C, D), jnp.float32),
        pltpu.VMEM((NUM_REL, D), jnp.float32),
        pltpu.VMEM((WROWS, C), jnp.float32),
        pltpu.SemaphoreType.DMA((2,)),
    ],
)
def _sc_kernel(*args):
    _sc_body(*args)


def kernel(embeddings, relation_weights, pos_src_interacts,
           pos_dst_interacts, neg_src_interacts, neg_dst_interacts,
           pos_src_regulates, pos_dst_regulates, neg_src_regulates,
           neg_dst_regulates):
    idx = [
        jnp.asarray(a, jnp.int32).reshape(NW, WROWS, C)
        for a in (pos_src_interacts, pos_dst_interacts,
                  neg_src_interacts, neg_dst_interacts,
                  pos_src_regulates, pos_dst_regulates,
                  neg_src_regulates, neg_dst_regulates)
    ]
    out = _sc_kernel(embeddings, relation_weights, *idx)
    return out.reshape(4, E)
